# baseline (speedup 1.0000x reference)
import jax
import jax.numpy as jnp
from jax import lax
from jax.experimental import pallas as pl
from jax.experimental.pallas import tpu as pltpu
from jax.experimental.pallas import tpu_sc as plsc

_BATCH = 16384
_D = 32
_NROWS = 1000000
_NC = 2
_NS = 16
_NW = _NC * _NS
_BPW = _BATCH // _NW
_CHUNK = 128
_NCHUNK = _BPW // _CHUNK
_L = 16
_WAVE = 4


def _mf_body(uid_hbm, iid_hbm, pf_hbm, qf_hbm, ub_hbm, ib_hbm, out_hbm,
             uidx, iidx, idxu, idxi, pbuf, qbuf, bu_v, bi_v, out_v,
             sem0, sem1, semb):
    wid = lax.axis_index("s") * _NC + lax.axis_index("c")
    sems = (sem0, sem1)
    pltpu.sync_copy(uid_hbm.at[wid], uidx)
    pltpu.sync_copy(iid_hbm.at[wid], iidx)

    bias_cps = []
    for c in range(_NCHUNK):
        sl = pl.ds(c * _CHUNK, _CHUNK)
        bias_cps.append(pltpu.async_copy(ub_hbm.at[uidx.at[c]], bu_v.at[sl], semb))
        bias_cps.append(pltpu.async_copy(ib_hbm.at[iidx.at[c]], bi_v.at[sl], semb))

    def build(c, carry):
        cb = c * _L
        chunk = c // (_CHUNK // _L)
        off = cb - chunk * _CHUNK
        uv = uidx[chunk, pl.ds(off, _L)]
        iv = iidx[chunk, pl.ds(off, _L)]
        for k in range(_D):
            idxu[k, pl.ds(cb, _L)] = uv + k * _NROWS
            idxi[k, pl.ds(cb, _L)] = iv + k * _NROWS
        return carry

    lax.fori_loop(0, _BPW // _L, build, 0)

    def fire_wave(w):
        cps = []
        for k in range(w * _WAVE, (w + 1) * _WAVE):
            for c in range(_NCHUNK):
                sl = pl.ds(c * _CHUNK, _CHUNK)
                cps.append(pltpu.async_copy(
                    pf_hbm.at[idxu.at[k, sl]], pbuf.at[k, sl], sems[w % 2]))
                cps.append(pltpu.async_copy(
                    qf_hbm.at[idxi.at[k, sl]], qbuf.at[k, sl], sems[w % 2]))
        return cps

    nwaves = _D // _WAVE
    pending = fire_wave(0)
    for w in range(1, nwaves + 1):
        nxt = fire_wave(w) if w < nwaves else []
        for cp in pending:
            cp.wait()
        pending = nxt
    for cp in bias_cps:
        cp.wait()

    def group(g, carry):
        gb = g * _L
        acc = bu_v[pl.ds(gb, _L)] + bi_v[pl.ds(gb, _L)]
        for k in range(_D):
            acc = acc + pbuf[k, pl.ds(gb, _L)] * qbuf[k, pl.ds(gb, _L)]
        out_v[pl.ds(gb, _L)] = acc
        return carry

    lax.fori_loop(0, _BPW // _L, group, 0)

    pltpu.sync_copy(out_v, out_hbm.at[pl.ds(wid * _BPW, _BPW)])


@jax.jit
def _mf(uid3, iid3, PF, QF, ub, ib):
    mesh = plsc.VectorSubcoreMesh(core_axis_name="c", subcore_axis_name="s")
    return pl.kernel(
        _mf_body,
        mesh=mesh,
        compiler_params=pltpu.CompilerParams(needs_layout_passes=False),
        out_type=jax.ShapeDtypeStruct((_BATCH,), jnp.float32),
        scratch_types=[
            pltpu.VMEM((_NCHUNK, _CHUNK), jnp.int32),
            pltpu.VMEM((_NCHUNK, _CHUNK), jnp.int32),
            pltpu.VMEM((_D, _BPW), jnp.int32),
            pltpu.VMEM((_D, _BPW), jnp.int32),
            pltpu.VMEM((_D, _BPW), jnp.float32),
            pltpu.VMEM((_D, _BPW), jnp.float32),
            pltpu.VMEM((_BPW,), jnp.float32),
            pltpu.VMEM((_BPW,), jnp.float32),
            pltpu.VMEM((_BPW,), jnp.float32),
            pltpu.SemaphoreType.DMA,
            pltpu.SemaphoreType.DMA,
            pltpu.SemaphoreType.DMA,
        ],
    )(uid3, iid3, PF, QF, ub, ib)


def kernel(user_id, item_id, P, Q, user_bias, item_bias):
    shape3 = (_NW, _NCHUNK, _CHUNK)
    uid3 = user_id.reshape(shape3)
    iid3 = item_id.reshape(shape3)
    PF = P.T.reshape(-1)
    QF = Q.T.reshape(-1)
    ub = user_bias.reshape(-1)
    ib = item_bias.reshape(-1)
    return _mf(uid3, iid3, PF, QF, ub, ib)
